# baseline (speedup 1.0000x reference)
import functools

import jax
import jax.numpy as jnp
from jax import lax
from jax.experimental import pallas as pl
from jax.experimental.pallas import tpu as pltpu
from jax.experimental.pallas import tpu_sc as plsc

B = 16384
IN_FEATURES = 64
ED = 10
HD = 32
CHUNK = 128
NC = 2
NS = 16
NW = NC * NS


WPT = B * ED
NCH = WPT // (CHUNK * NW)
GRP = 4
NSTEP = NCH // GRP


def _make_sc_gather():
    mesh = plsc.VectorSubcoreMesh(
        core_axis_name="c", subcore_axis_name="s", num_cores=NC,
        num_subcores=NS)

    @functools.partial(
        pl.kernel,
        mesh=mesh,
        compiler_params=pltpu.CompilerParams(use_tc_tiling_on_sc=False),
        out_type=[
            jax.ShapeDtypeStruct((WPT // CHUNK, CHUNK), jnp.float32),
            jax.ShapeDtypeStruct((WPT // CHUNK, CHUNK), jnp.float32),
        ],
        scratch_types=[
            pltpu.VMEM((NCH, CHUNK), jnp.int32),
            pltpu.VMEM((NCH, CHUNK), jnp.int32),
            pltpu.VMEM((NCH, CHUNK), jnp.float32),
            pltpu.VMEM((NCH, CHUNK), jnp.float32),
            pltpu.SemaphoreType.DMA,
        ],
    )
    def sc_gather(bidx_hbm, zidx_hbm, btab_hbm, ztab_hbm, be_out, ze_out,
                  bidx_v, zidx_v, bw_v, zw_v, sem):
        wid = lax.axis_index("s") * NC + lax.axis_index("c")
        base = wid * NCH
        pltpu.sync_copy(bidx_hbm.at[pl.ds(base, NCH)], bidx_v)
        pltpu.sync_copy(zidx_hbm.at[pl.ds(base, NCH)], zidx_v)

        def step(g, carry):
            copies = []
            for j in range(GRP):
                c = g * GRP + j
                copies.append(
                    pltpu.async_copy(btab_hbm.at[bidx_v.at[c]], bw_v.at[c], sem))
                copies.append(
                    pltpu.async_copy(ztab_hbm.at[zidx_v.at[c]], zw_v.at[c], sem))
            for cp in copies:
                cp.wait()
            return carry

        lax.fori_loop(0, NSTEP, step, 0)
        pltpu.sync_copy(bw_v, be_out.at[pl.ds(base, NCH)])
        pltpu.sync_copy(zw_v, ze_out.at[pl.ds(base, NCH)])

    return sc_gather


def _mlp_body(be_ref, ze_ref, x_ref, w1a_ref, w1b_ref, w1c_ref, b1_ref,
              w2_ref, b2_ref, w3_ref, b3_ref, o_ref):
    h = (
        jnp.dot(be_ref[...], w1a_ref[...], preferred_element_type=jnp.float32)
        + jnp.dot(ze_ref[...], w1b_ref[...], preferred_element_type=jnp.float32)
        + jnp.dot(x_ref[...], w1c_ref[...], preferred_element_type=jnp.float32)
        + b1_ref[...]
    )
    h = jnp.maximum(h, 0.0)
    h = jnp.dot(h, w2_ref[...], preferred_element_type=jnp.float32) + b2_ref[...]
    h = jnp.maximum(h, 0.0)
    o_ref[...] = (
        jnp.dot(h, w3_ref[...], preferred_element_type=jnp.float32) + b3_ref[...]
    )


def kernel(brand_tensor, zip_tensor, input_tensor, brand_table, zip_table,
           W1, b1, W2, b2, W3, b3):
    nb = brand_table.shape[0]
    nz = zip_table.shape[0]
    bflat = brand_table.T.reshape(-1)
    zflat = zip_table.T.reshape(-1)
    col = jnp.arange(ED, dtype=jnp.int32)
    bwidx = (brand_tensor[:, None] + col * nb).reshape(WPT // CHUNK, CHUNK)
    zwidx = (zip_tensor[:, None] + col * nz).reshape(WPT // CHUNK, CHUNK)
    bew, zew = _make_sc_gather()(bwidx, zwidx, bflat, zflat)
    be = bew.reshape(B, ED)
    ze = zew.reshape(B, ED)

    w1a = W1[:ED]
    w1b = W1[ED:2 * ED]
    w1c = W1[2 * ED:]
    b1_2d = b1.reshape(1, -1)
    b2_2d = b2.reshape(1, -1)
    b3_2d = b3.reshape(1, -1)

    blk = 2048
    h1 = HD * 2
    out = pl.pallas_call(
        _mlp_body,
        grid=(B // blk,),
        in_specs=[
            pl.BlockSpec((blk, ED), lambda i: (i, 0)),
            pl.BlockSpec((blk, ED), lambda i: (i, 0)),
            pl.BlockSpec((blk, IN_FEATURES), lambda i: (i, 0)),
            pl.BlockSpec((ED, h1), lambda i: (0, 0)),
            pl.BlockSpec((ED, h1), lambda i: (0, 0)),
            pl.BlockSpec((IN_FEATURES, h1), lambda i: (0, 0)),
            pl.BlockSpec((1, h1), lambda i: (0, 0)),
            pl.BlockSpec((h1, HD), lambda i: (0, 0)),
            pl.BlockSpec((1, HD), lambda i: (0, 0)),
            pl.BlockSpec((HD, 1), lambda i: (0, 0)),
            pl.BlockSpec((1, 1), lambda i: (0, 0)),
        ],
        out_specs=pl.BlockSpec((blk, 1), lambda i: (i, 0)),
        out_shape=jax.ShapeDtypeStruct((B, 1), jnp.float32),
    )(be, ze, input_tensor, w1a, w1b, w1c, b1_2d, W2, b2_2d, W3, b3_2d)
    return out


# speedup vs baseline: 4.3533x; 4.3533x over previous
import functools

import jax
import jax.numpy as jnp
from jax import lax
from jax.experimental import pallas as pl
from jax.experimental.pallas import tpu as pltpu
from jax.experimental.pallas import tpu_sc as plsc

B = 16384
IN_FEATURES = 64
ED = 10
HD = 32
CHUNK = 128
NC = 2
NS = 16
NW = NC * NS


WPT = B * ED
NCH = WPT // (CHUNK * NW)
GRP = 4
NSTEP = NCH // GRP


NB = 1000000
NZ = 100000
WPT = B * ED
NCH = WPT // (CHUNK * NW)
GRP = 4
NSTEP = NCH // GRP
CW = 7680
BSTEPS = 4
B_EXTRA = ((983040, 7680), (990720, 7680), (998400, 1536))
ZW = 3072
Z_EXTRA = ((NW * ZW, 1664),)
BT_RAG = 64
ZT_RAG = 32


def _make_sc_flatten():
    mesh = plsc.VectorSubcoreMesh(
        core_axis_name="c", subcore_axis_name="s", num_cores=NC,
        num_subcores=NS)

    @functools.partial(
        pl.kernel,
        mesh=mesh,
        compiler_params=pltpu.CompilerParams(use_tc_tiling_on_sc=True),
        out_type=[
            jax.ShapeDtypeStruct((NB * ED,), jnp.float32),
            jax.ShapeDtypeStruct((NZ * ED,), jnp.float32),
        ],
        scratch_types=[
            pltpu.VMEM((ED, CW), jnp.float32),
            pltpu.VMEM((CW,), jnp.float32),
            pltpu.SemaphoreType.DMA,
            pltpu.SemaphoreType.DMA,
        ],
    )
    def sc_flatten(btabt_hbm, ztabt_hbm, btail_hbm, ztail_hbm,
                   bflat_out, zflat_out, ibuf, obuf, isem, osem):
        wid = lax.axis_index("s") * NC + lax.axis_index("c")

        def destride_col(c, ln):
            def vcopy(w, carry):
                obuf[pl.ds(w * 128, 128)] = ibuf[c, pl.ds(w * 128, 128)]
                return carry
            lax.fori_loop(0, ln // 128, vcopy, 0)

        def stage(tab, src_off, ln, flat, stride, dst_off):
            pltpu.async_copy(
                tab.at[:, pl.ds(src_off, ln)], ibuf.at[:, pl.ds(0, ln)],
                isem).wait()
            for c in range(ED):
                destride_col(c, ln)
                pltpu.async_copy(
                    obuf.at[pl.ds(0, ln)],
                    flat.at[pl.ds(c * stride + dst_off, ln)], osem).wait()

        for s in range(BSTEPS):
            off = wid * BSTEPS * CW + s * CW
            stage(btabt_hbm, off, CW, bflat_out, NB, off)
        stage(ztabt_hbm, wid * ZW, ZW, zflat_out, NZ, wid * ZW)

        @pl.when(wid == 0)
        def _tails():
            for off, ln in B_EXTRA:
                stage(btabt_hbm, off, ln, bflat_out, NB, off)
            for off, ln in Z_EXTRA:
                stage(ztabt_hbm, off, ln, zflat_out, NZ, off)
            for c in range(ED):
                pltpu.async_copy(
                    btail_hbm.at[pl.ds(c * BT_RAG, BT_RAG)],
                    obuf.at[pl.ds(0, BT_RAG)], isem).wait()
                pltpu.async_copy(
                    obuf.at[pl.ds(0, BT_RAG)],
                    bflat_out.at[pl.ds(c * NB + NB - BT_RAG, BT_RAG)],
                    osem).wait()
                pltpu.async_copy(
                    ztail_hbm.at[pl.ds(c * ZT_RAG, ZT_RAG)],
                    obuf.at[pl.ds(0, ZT_RAG)], isem).wait()
                pltpu.async_copy(
                    obuf.at[pl.ds(0, ZT_RAG)],
                    zflat_out.at[pl.ds(c * NZ + NZ - ZT_RAG, ZT_RAG)],
                    osem).wait()

    return sc_flatten


def _make_sc_gather():
    mesh = plsc.VectorSubcoreMesh(
        core_axis_name="c", subcore_axis_name="s", num_cores=NC,
        num_subcores=NS)

    @functools.partial(
        pl.kernel,
        mesh=mesh,
        compiler_params=pltpu.CompilerParams(use_tc_tiling_on_sc=False),
        out_type=[
            jax.ShapeDtypeStruct((WPT // CHUNK, CHUNK), jnp.float32),
            jax.ShapeDtypeStruct((WPT // CHUNK, CHUNK), jnp.float32),
        ],
        scratch_types=[
            pltpu.VMEM((NCH, CHUNK), jnp.int32),
            pltpu.VMEM((NCH, CHUNK), jnp.int32),
            pltpu.VMEM((NCH, CHUNK), jnp.float32),
            pltpu.VMEM((NCH, CHUNK), jnp.float32),
            pltpu.SemaphoreType.DMA,
        ],
    )
    def sc_gather(bidx_hbm, zidx_hbm, btab_hbm, ztab_hbm, be_out, ze_out,
                  bidx_v, zidx_v, bw_v, zw_v, sem):
        wid = lax.axis_index("s") * NC + lax.axis_index("c")
        base = wid * NCH
        pltpu.sync_copy(bidx_hbm.at[pl.ds(base, NCH)], bidx_v)
        pltpu.sync_copy(zidx_hbm.at[pl.ds(base, NCH)], zidx_v)

        def step(g, carry):
            copies = []
            for j in range(GRP):
                c = g * GRP + j
                copies.append(
                    pltpu.async_copy(btab_hbm.at[bidx_v.at[c]], bw_v.at[c], sem))
                copies.append(
                    pltpu.async_copy(ztab_hbm.at[zidx_v.at[c]], zw_v.at[c], sem))
            for cp in copies:
                cp.wait()
            return carry

        lax.fori_loop(0, NSTEP, step, 0)
        pltpu.sync_copy(bw_v, be_out.at[pl.ds(base, NCH)])
        pltpu.sync_copy(zw_v, ze_out.at[pl.ds(base, NCH)])

    return sc_gather


def _mlp_body(be_ref, ze_ref, x_ref, w1a_ref, w1b_ref, w1c_ref, b1_ref,
              w2_ref, b2_ref, w3_ref, b3_ref, o_ref):
    h = (
        jnp.dot(be_ref[...], w1a_ref[...], preferred_element_type=jnp.float32)
        + jnp.dot(ze_ref[...], w1b_ref[...], preferred_element_type=jnp.float32)
        + jnp.dot(x_ref[...], w1c_ref[...], preferred_element_type=jnp.float32)
        + b1_ref[...]
    )
    h = jnp.maximum(h, 0.0)
    h = jnp.dot(h, w2_ref[...], preferred_element_type=jnp.float32) + b2_ref[...]
    h = jnp.maximum(h, 0.0)
    o_ref[...] = (
        jnp.dot(h, w3_ref[...], preferred_element_type=jnp.float32) + b3_ref[...]
    )


def kernel(brand_tensor, zip_tensor, input_tensor, brand_table, zip_table,
           W1, b1, W2, b2, W3, b3):
    btail = brand_table.T[:, NB - BT_RAG:].reshape(-1)
    ztail = zip_table.T[:, NZ - ZT_RAG:].reshape(-1)
    bflat, zflat = _make_sc_flatten()(brand_table.T, zip_table.T, btail,
                                      ztail)
    col = jnp.arange(ED, dtype=jnp.int32)
    bwidx = (brand_tensor[:, None] + col * NB).reshape(WPT // CHUNK, CHUNK)
    zwidx = (zip_tensor[:, None] + col * NZ).reshape(WPT // CHUNK, CHUNK)
    bew, zew = _make_sc_gather()(bwidx, zwidx, bflat, zflat)
    be = bew.reshape(B, ED)
    ze = zew.reshape(B, ED)

    w1a = W1[:ED]
    w1b = W1[ED:2 * ED]
    w1c = W1[2 * ED:]
    b1_2d = b1.reshape(1, -1)
    b2_2d = b2.reshape(1, -1)
    b3_2d = b3.reshape(1, -1)

    blk = 2048
    h1 = HD * 2
    out = pl.pallas_call(
        _mlp_body,
        grid=(B // blk,),
        in_specs=[
            pl.BlockSpec((blk, ED), lambda i: (i, 0)),
            pl.BlockSpec((blk, ED), lambda i: (i, 0)),
            pl.BlockSpec((blk, IN_FEATURES), lambda i: (i, 0)),
            pl.BlockSpec((ED, h1), lambda i: (0, 0)),
            pl.BlockSpec((ED, h1), lambda i: (0, 0)),
            pl.BlockSpec((IN_FEATURES, h1), lambda i: (0, 0)),
            pl.BlockSpec((1, h1), lambda i: (0, 0)),
            pl.BlockSpec((h1, HD), lambda i: (0, 0)),
            pl.BlockSpec((1, HD), lambda i: (0, 0)),
            pl.BlockSpec((HD, 1), lambda i: (0, 0)),
            pl.BlockSpec((1, 1), lambda i: (0, 0)),
        ],
        out_specs=pl.BlockSpec((blk, 1), lambda i: (i, 0)),
        out_shape=jax.ShapeDtypeStruct((B, 1), jnp.float32),
    )(be, ze, input_tensor, w1a, w1b, w1c, b1_2d, W2, b2_2d, W3, b3_2d)
    return out


# speedup vs baseline: 5.1010x; 1.1718x over previous
import functools

import jax
import jax.numpy as jnp
from jax import lax
from jax.experimental import pallas as pl
from jax.experimental.pallas import tpu as pltpu
from jax.experimental.pallas import tpu_sc as plsc

B = 16384
IN_FEATURES = 64
ED = 10
HD = 32
CHUNK = 128
NC = 2
NS = 16
NW = NC * NS


WPT = B * ED
NCH = WPT // (CHUNK * NW)
GRP = 4
NSTEP = NCH // GRP


NB = 1000000
NZ = 100000
WPT = B * ED
NCH = WPT // (CHUNK * NW)
GRP = 4
NSTEP = NCH // GRP
CW = 7680
BSTEPS = 4
B_EXTRA = ((983040, 7680), (990720, 7680), (998400, 1536))
ZW = 3072
Z_EXTRA = ((NW * ZW, 1664),)
BT_RAG = 64
ZT_RAG = 32


def _make_sc_flatten():
    mesh = plsc.VectorSubcoreMesh(
        core_axis_name="c", subcore_axis_name="s", num_cores=NC,
        num_subcores=NS)

    @functools.partial(
        pl.kernel,
        mesh=mesh,
        compiler_params=pltpu.CompilerParams(use_tc_tiling_on_sc=True),
        out_type=[
            jax.ShapeDtypeStruct((NB * ED,), jnp.float32),
            jax.ShapeDtypeStruct((NZ * ED,), jnp.float32),
        ],
        scratch_types=[
            pltpu.VMEM((ED, CW), jnp.float32),
            pltpu.VMEM((CW,), jnp.float32),
            pltpu.SemaphoreType.DMA,
            pltpu.SemaphoreType.DMA,
        ],
    )
    def sc_flatten(btabt_hbm, ztabt_hbm, btail_hbm, ztail_hbm,
                   bflat_out, zflat_out, ibuf, obuf, isem, osem):
        wid = lax.axis_index("s") * NC + lax.axis_index("c")

        def destride_col(c, ln):
            def vcopy(w, carry):
                obuf[pl.ds(w * 128, 128)] = ibuf[c, pl.ds(w * 128, 128)]
                return carry
            lax.fori_loop(0, ln // 128, vcopy, 0)

        def stage(tab, src_off, ln, flat, stride, dst_off):
            pltpu.async_copy(
                tab.at[:, pl.ds(src_off, ln)], ibuf.at[:, pl.ds(0, ln)],
                isem).wait()
            for c in range(ED):
                destride_col(c, ln)
                pltpu.async_copy(
                    obuf.at[pl.ds(0, ln)],
                    flat.at[pl.ds(c * stride + dst_off, ln)], osem).wait()

        for s in range(BSTEPS):
            off = wid * BSTEPS * CW + s * CW
            stage(btabt_hbm, off, CW, bflat_out, NB, off)
        stage(ztabt_hbm, wid * ZW, ZW, zflat_out, NZ, wid * ZW)

        for i, (off, ln) in enumerate(B_EXTRA):
            @pl.when(wid == i)
            def _bextra(off=off, ln=ln):
                stage(btabt_hbm, off, ln, bflat_out, NB, off)
        for i, (off, ln) in enumerate(Z_EXTRA):
            @pl.when(wid == len(B_EXTRA) + i)
            def _zextra(off=off, ln=ln):
                stage(ztabt_hbm, off, ln, zflat_out, NZ, off)

        @pl.when(wid == len(B_EXTRA) + len(Z_EXTRA))
        def _tails():
            for c in range(ED):
                pltpu.async_copy(
                    btail_hbm.at[pl.ds(c * BT_RAG, BT_RAG)],
                    obuf.at[pl.ds(0, BT_RAG)], isem).wait()
                pltpu.async_copy(
                    obuf.at[pl.ds(0, BT_RAG)],
                    bflat_out.at[pl.ds(c * NB + NB - BT_RAG, BT_RAG)],
                    osem).wait()
                pltpu.async_copy(
                    ztail_hbm.at[pl.ds(c * ZT_RAG, ZT_RAG)],
                    obuf.at[pl.ds(0, ZT_RAG)], isem).wait()
                pltpu.async_copy(
                    obuf.at[pl.ds(0, ZT_RAG)],
                    zflat_out.at[pl.ds(c * NZ + NZ - ZT_RAG, ZT_RAG)],
                    osem).wait()

    return sc_flatten


def _make_sc_gather():
    mesh = plsc.VectorSubcoreMesh(
        core_axis_name="c", subcore_axis_name="s", num_cores=NC,
        num_subcores=NS)

    @functools.partial(
        pl.kernel,
        mesh=mesh,
        compiler_params=pltpu.CompilerParams(use_tc_tiling_on_sc=False),
        out_type=[
            jax.ShapeDtypeStruct((WPT // CHUNK, CHUNK), jnp.float32),
            jax.ShapeDtypeStruct((WPT // CHUNK, CHUNK), jnp.float32),
        ],
        scratch_types=[
            pltpu.VMEM((NCH, CHUNK), jnp.int32),
            pltpu.VMEM((NCH, CHUNK), jnp.int32),
            pltpu.VMEM((NCH, CHUNK), jnp.float32),
            pltpu.VMEM((NCH, CHUNK), jnp.float32),
            pltpu.SemaphoreType.DMA,
        ],
    )
    def sc_gather(bidx_hbm, zidx_hbm, btab_hbm, ztab_hbm, be_out, ze_out,
                  bidx_v, zidx_v, bw_v, zw_v, sem):
        wid = lax.axis_index("s") * NC + lax.axis_index("c")
        base = wid * NCH
        pltpu.sync_copy(bidx_hbm.at[pl.ds(base, NCH)], bidx_v)
        pltpu.sync_copy(zidx_hbm.at[pl.ds(base, NCH)], zidx_v)

        def step(g, carry):
            copies = []
            for j in range(GRP):
                c = g * GRP + j
                copies.append(
                    pltpu.async_copy(btab_hbm.at[bidx_v.at[c]], bw_v.at[c], sem))
                copies.append(
                    pltpu.async_copy(ztab_hbm.at[zidx_v.at[c]], zw_v.at[c], sem))
            for cp in copies:
                cp.wait()
            return carry

        lax.fori_loop(0, NSTEP, step, 0)
        pltpu.sync_copy(bw_v, be_out.at[pl.ds(base, NCH)])
        pltpu.sync_copy(zw_v, ze_out.at[pl.ds(base, NCH)])

    return sc_gather


def _mlp_body(be_ref, ze_ref, x_ref, w1a_ref, w1b_ref, w1c_ref, b1_ref,
              w2_ref, b2_ref, w3_ref, b3_ref, o_ref):
    h = (
        jnp.dot(be_ref[...], w1a_ref[...], preferred_element_type=jnp.float32)
        + jnp.dot(ze_ref[...], w1b_ref[...], preferred_element_type=jnp.float32)
        + jnp.dot(x_ref[...], w1c_ref[...], preferred_element_type=jnp.float32)
        + b1_ref[...]
    )
    h = jnp.maximum(h, 0.0)
    h = jnp.dot(h, w2_ref[...], preferred_element_type=jnp.float32) + b2_ref[...]
    h = jnp.maximum(h, 0.0)
    o_ref[...] = (
        jnp.dot(h, w3_ref[...], preferred_element_type=jnp.float32) + b3_ref[...]
    )


def kernel(brand_tensor, zip_tensor, input_tensor, brand_table, zip_table,
           W1, b1, W2, b2, W3, b3):
    btail = brand_table.T[:, NB - BT_RAG:].reshape(-1)
    ztail = zip_table.T[:, NZ - ZT_RAG:].reshape(-1)
    bflat, zflat = _make_sc_flatten()(brand_table.T, zip_table.T, btail,
                                      ztail)
    col = jnp.arange(ED, dtype=jnp.int32)
    bwidx = (brand_tensor[:, None] + col * NB).reshape(WPT // CHUNK, CHUNK)
    zwidx = (zip_tensor[:, None] + col * NZ).reshape(WPT // CHUNK, CHUNK)
    bew, zew = _make_sc_gather()(bwidx, zwidx, bflat, zflat)
    be = bew.reshape(B, ED)
    ze = zew.reshape(B, ED)

    w1a = W1[:ED]
    w1b = W1[ED:2 * ED]
    w1c = W1[2 * ED:]
    b1_2d = b1.reshape(1, -1)
    b2_2d = b2.reshape(1, -1)
    b3_2d = b3.reshape(1, -1)

    blk = 2048
    h1 = HD * 2
    out = pl.pallas_call(
        _mlp_body,
        grid=(B // blk,),
        in_specs=[
            pl.BlockSpec((blk, ED), lambda i: (i, 0)),
            pl.BlockSpec((blk, ED), lambda i: (i, 0)),
            pl.BlockSpec((blk, IN_FEATURES), lambda i: (i, 0)),
            pl.BlockSpec((ED, h1), lambda i: (0, 0)),
            pl.BlockSpec((ED, h1), lambda i: (0, 0)),
            pl.BlockSpec((IN_FEATURES, h1), lambda i: (0, 0)),
            pl.BlockSpec((1, h1), lambda i: (0, 0)),
            pl.BlockSpec((h1, HD), lambda i: (0, 0)),
            pl.BlockSpec((1, HD), lambda i: (0, 0)),
            pl.BlockSpec((HD, 1), lambda i: (0, 0)),
            pl.BlockSpec((1, 1), lambda i: (0, 0)),
        ],
        out_specs=pl.BlockSpec((blk, 1), lambda i: (i, 0)),
        out_shape=jax.ShapeDtypeStruct((B, 1), jnp.float32),
    )(be, ze, input_tensor, w1a, w1b, w1c, b1_2d, W2, b2_2d, W3, b3_2d)
    return out


# speedup vs baseline: 5.7755x; 1.1322x over previous
import functools

import jax
import jax.numpy as jnp
from jax import lax
from jax.experimental import pallas as pl
from jax.experimental.pallas import tpu as pltpu
from jax.experimental.pallas import tpu_sc as plsc

B = 16384
IN_FEATURES = 64
ED = 10
HD = 32
CHUNK = 128
NC = 2
NS = 16
NW = NC * NS


WPT = B * ED
NCH = WPT // (CHUNK * NW)
GRP = 4
NSTEP = NCH // GRP


NB = 1000000
NZ = 100000
WPT = B * ED
NCH = WPT // (CHUNK * NW)
GRP = 4
NSTEP = NCH // GRP
CW = 3840
BSTEPS = 8
B_EXTRA = ((983040, 3840), (986880, 3840), (990720, 3840), (994560, 3840),
           (998400, 1536))
ZW = 3072
Z_EXTRA = ((NW * ZW, 1664),)
BT_RAG = 64
ZT_RAG = 32


def _make_sc_flatten():
    mesh = plsc.VectorSubcoreMesh(
        core_axis_name="c", subcore_axis_name="s", num_cores=NC,
        num_subcores=NS)

    @functools.partial(
        pl.kernel,
        mesh=mesh,
        compiler_params=pltpu.CompilerParams(use_tc_tiling_on_sc=True),
        out_type=[
            jax.ShapeDtypeStruct((NB * ED,), jnp.float32),
            jax.ShapeDtypeStruct((NZ * ED,), jnp.float32),
        ],
        scratch_types=[
            pltpu.VMEM((2, ED, CW), jnp.float32),
            pltpu.VMEM((CW,), jnp.float32),
            pltpu.VMEM((CW,), jnp.float32),
            pltpu.SemaphoreType.DMA,
            pltpu.SemaphoreType.DMA,
        ],
    )
    def sc_flatten(btabt_hbm, ztabt_hbm, btail_hbm, ztail_hbm,
                   bflat_out, zflat_out, ibuf, obuf0, obuf1, isem, osem):
        wid = lax.axis_index("s") * NC + lax.axis_index("c")
        obufs = (obuf0, obuf1)

        def destride_col(slot, c, ob, ln):
            def vcopy(w, carry):
                ob[pl.ds(w * 128, 128)] = ibuf[slot, c, pl.ds(w * 128, 128)]
                return carry
            lax.fori_loop(0, ln // 128, vcopy, 0)

        def run_jobs(jobs):
            ins = [pltpu.async_copy(
                jobs[0][0].at[:, pl.ds(jobs[0][1], jobs[0][2])],
                ibuf.at[0].at[:, pl.ds(0, jobs[0][2])], isem)]
            outs = []
            for t, (tab, off, ln, flat, stride) in enumerate(jobs):
                if t + 1 < len(jobs):
                    nt, noff, nln = jobs[t + 1][0], jobs[t + 1][1], jobs[t + 1][2]
                    ins.append(pltpu.async_copy(
                        nt.at[:, pl.ds(noff, nln)],
                        ibuf.at[(t + 1) % 2].at[:, pl.ds(0, nln)], isem))
                ins[t].wait()
                for c in range(ED):
                    g = t * ED + c
                    if g >= 2:
                        outs[g - 2].wait()
                    destride_col(t % 2, c, obufs[g % 2], ln)
                    outs.append(pltpu.async_copy(
                        obufs[g % 2].at[pl.ds(0, ln)],
                        flat.at[pl.ds(c * stride + off, ln)], osem))
            for o in outs[-2:]:
                o.wait()

        def stage(tab, src_off, ln, flat, stride, dst_off):
            run_jobs([(tab, src_off, ln, flat, stride)])

        bbase = wid * BSTEPS * CW
        run_jobs([(btabt_hbm, bbase + s * CW, CW, bflat_out, NB)
                  for s in range(BSTEPS)])
        run_jobs([(ztabt_hbm, wid * ZW, ZW, zflat_out, NZ)])

        for i, (off, ln) in enumerate(B_EXTRA):
            @pl.when(wid == i)
            def _bextra(off=off, ln=ln):
                stage(btabt_hbm, off, ln, bflat_out, NB, off)
        for i, (off, ln) in enumerate(Z_EXTRA):
            @pl.when(wid == len(B_EXTRA) + i)
            def _zextra(off=off, ln=ln):
                stage(ztabt_hbm, off, ln, zflat_out, NZ, off)

        @pl.when(wid == len(B_EXTRA) + len(Z_EXTRA))
        def _tails():
            for c in range(ED):
                pltpu.async_copy(
                    btail_hbm.at[pl.ds(c * BT_RAG, BT_RAG)],
                    obuf0.at[pl.ds(0, BT_RAG)], isem).wait()
                pltpu.async_copy(
                    obuf0.at[pl.ds(0, BT_RAG)],
                    bflat_out.at[pl.ds(c * NB + NB - BT_RAG, BT_RAG)],
                    osem).wait()
                pltpu.async_copy(
                    ztail_hbm.at[pl.ds(c * ZT_RAG, ZT_RAG)],
                    obuf0.at[pl.ds(0, ZT_RAG)], isem).wait()
                pltpu.async_copy(
                    obuf0.at[pl.ds(0, ZT_RAG)],
                    zflat_out.at[pl.ds(c * NZ + NZ - ZT_RAG, ZT_RAG)],
                    osem).wait()

    return sc_flatten


def _make_sc_gather():
    mesh = plsc.VectorSubcoreMesh(
        core_axis_name="c", subcore_axis_name="s", num_cores=NC,
        num_subcores=NS)

    @functools.partial(
        pl.kernel,
        mesh=mesh,
        compiler_params=pltpu.CompilerParams(use_tc_tiling_on_sc=False),
        out_type=[
            jax.ShapeDtypeStruct((WPT // CHUNK, CHUNK), jnp.float32),
            jax.ShapeDtypeStruct((WPT // CHUNK, CHUNK), jnp.float32),
        ],
        scratch_types=[
            pltpu.VMEM((NCH, CHUNK), jnp.int32),
            pltpu.VMEM((NCH, CHUNK), jnp.int32),
            pltpu.VMEM((NCH, CHUNK), jnp.float32),
            pltpu.VMEM((NCH, CHUNK), jnp.float32),
            pltpu.SemaphoreType.DMA,
        ],
    )
    def sc_gather(bidx_hbm, zidx_hbm, btab_hbm, ztab_hbm, be_out, ze_out,
                  bidx_v, zidx_v, bw_v, zw_v, sem):
        wid = lax.axis_index("s") * NC + lax.axis_index("c")
        base = wid * NCH
        pltpu.sync_copy(bidx_hbm.at[pl.ds(base, NCH)], bidx_v)
        pltpu.sync_copy(zidx_hbm.at[pl.ds(base, NCH)], zidx_v)

        def step(g, carry):
            copies = []
            for j in range(GRP):
                c = g * GRP + j
                copies.append(
                    pltpu.async_copy(btab_hbm.at[bidx_v.at[c]], bw_v.at[c], sem))
                copies.append(
                    pltpu.async_copy(ztab_hbm.at[zidx_v.at[c]], zw_v.at[c], sem))
            for cp in copies:
                cp.wait()
            return carry

        lax.fori_loop(0, NSTEP, step, 0)
        pltpu.sync_copy(bw_v, be_out.at[pl.ds(base, NCH)])
        pltpu.sync_copy(zw_v, ze_out.at[pl.ds(base, NCH)])

    return sc_gather


def _mlp_body(be_ref, ze_ref, x_ref, w1a_ref, w1b_ref, w1c_ref, b1_ref,
              w2_ref, b2_ref, w3_ref, b3_ref, o_ref):
    h = (
        jnp.dot(be_ref[...], w1a_ref[...], preferred_element_type=jnp.float32)
        + jnp.dot(ze_ref[...], w1b_ref[...], preferred_element_type=jnp.float32)
        + jnp.dot(x_ref[...], w1c_ref[...], preferred_element_type=jnp.float32)
        + b1_ref[...]
    )
    h = jnp.maximum(h, 0.0)
    h = jnp.dot(h, w2_ref[...], preferred_element_type=jnp.float32) + b2_ref[...]
    h = jnp.maximum(h, 0.0)
    o_ref[...] = (
        jnp.dot(h, w3_ref[...], preferred_element_type=jnp.float32) + b3_ref[...]
    )


def kernel(brand_tensor, zip_tensor, input_tensor, brand_table, zip_table,
           W1, b1, W2, b2, W3, b3):
    btail = brand_table.T[:, NB - BT_RAG:].reshape(-1)
    ztail = zip_table.T[:, NZ - ZT_RAG:].reshape(-1)
    bflat, zflat = _make_sc_flatten()(brand_table.T, zip_table.T, btail,
                                      ztail)
    col = jnp.arange(ED, dtype=jnp.int32)
    bwidx = (brand_tensor[:, None] + col * NB).reshape(WPT // CHUNK, CHUNK)
    zwidx = (zip_tensor[:, None] + col * NZ).reshape(WPT // CHUNK, CHUNK)
    bew, zew = _make_sc_gather()(bwidx, zwidx, bflat, zflat)
    be = bew.reshape(B, ED)
    ze = zew.reshape(B, ED)

    w1a = W1[:ED]
    w1b = W1[ED:2 * ED]
    w1c = W1[2 * ED:]
    b1_2d = b1.reshape(1, -1)
    b2_2d = b2.reshape(1, -1)
    b3_2d = b3.reshape(1, -1)

    blk = 2048
    h1 = HD * 2
    out = pl.pallas_call(
        _mlp_body,
        grid=(B // blk,),
        in_specs=[
            pl.BlockSpec((blk, ED), lambda i: (i, 0)),
            pl.BlockSpec((blk, ED), lambda i: (i, 0)),
            pl.BlockSpec((blk, IN_FEATURES), lambda i: (i, 0)),
            pl.BlockSpec((ED, h1), lambda i: (0, 0)),
            pl.BlockSpec((ED, h1), lambda i: (0, 0)),
            pl.BlockSpec((IN_FEATURES, h1), lambda i: (0, 0)),
            pl.BlockSpec((1, h1), lambda i: (0, 0)),
            pl.BlockSpec((h1, HD), lambda i: (0, 0)),
            pl.BlockSpec((1, HD), lambda i: (0, 0)),
            pl.BlockSpec((HD, 1), lambda i: (0, 0)),
            pl.BlockSpec((1, 1), lambda i: (0, 0)),
        ],
        out_specs=pl.BlockSpec((blk, 1), lambda i: (i, 0)),
        out_shape=jax.ShapeDtypeStruct((B, 1), jnp.float32),
    )(be, ze, input_tensor, w1a, w1b, w1c, b1_2d, W2, b2_2d, W3, b3_2d)
    return out


# speedup vs baseline: 7.0220x; 1.2158x over previous
import functools

import jax
import jax.numpy as jnp
from jax import lax
from jax.experimental import pallas as pl
from jax.experimental.pallas import tpu as pltpu
from jax.experimental.pallas import tpu_sc as plsc

B = 16384
IN_FEATURES = 64
ED = 10
HD = 32
CHUNK = 128
NC = 2
NS = 16
NW = NC * NS


WPT = B * ED
NCH = WPT // (CHUNK * NW)
GRP = 4
NSTEP = NCH // GRP


NB = 1000000
NZ = 100000
WPT = B * ED
NCH = WPT // (CHUNK * NW)
GRP = 4
NSTEP = NCH // GRP
CW = 3840
BSTEPS = 8
B_EXTRA = ((983040, 3840), (986880, 3840), (990720, 3840), (994560, 3840),
           (998400, 1536))
ZW = 3072
Z_EXTRA = ((NW * ZW, 1664),)
BT_RAG = 64
ZT_RAG = 32


def _make_sc_flatten():
    mesh = plsc.VectorSubcoreMesh(
        core_axis_name="c", subcore_axis_name="s", num_cores=NC,
        num_subcores=NS)

    @functools.partial(
        pl.kernel,
        mesh=mesh,
        compiler_params=pltpu.CompilerParams(use_tc_tiling_on_sc=True),
        out_type=[
            jax.ShapeDtypeStruct((NB * ED,), jnp.float32),
            jax.ShapeDtypeStruct((NZ * ED,), jnp.float32),
        ],
        scratch_types=[
            pltpu.VMEM((2, ED, CW), jnp.float32),
            pltpu.VMEM((CW,), jnp.float32),
            pltpu.VMEM((CW,), jnp.float32),
            pltpu.SemaphoreType.DMA,
            pltpu.SemaphoreType.DMA,
        ],
    )
    def sc_flatten(btabt_hbm, ztabt_hbm, btail_hbm, ztail_hbm,
                   bflat_out, zflat_out, ibuf, obuf0, obuf1, isem, osem):
        wid = lax.axis_index("s") * NC + lax.axis_index("c")
        obufs = (obuf0, obuf1)

        def destride_col(slot, c, ob, ln):
            def vcopy(w, carry):
                ob[pl.ds(w * 128, 128)] = ibuf[slot, c, pl.ds(w * 128, 128)]
                return carry
            lax.fori_loop(0, ln // 128, vcopy, 0)

        def run_jobs(jobs):
            ins = [pltpu.async_copy(
                jobs[0][0].at[:, pl.ds(jobs[0][1], jobs[0][2])],
                ibuf.at[0].at[:, pl.ds(0, jobs[0][2])], isem)]
            outs = []
            for t, (tab, off, ln, flat, stride) in enumerate(jobs):
                if t + 1 < len(jobs):
                    nt, noff, nln = jobs[t + 1][0], jobs[t + 1][1], jobs[t + 1][2]
                    ins.append(pltpu.async_copy(
                        nt.at[:, pl.ds(noff, nln)],
                        ibuf.at[(t + 1) % 2].at[:, pl.ds(0, nln)], isem))
                ins[t].wait()
                for c in range(ED):
                    g = t * ED + c
                    if g >= 2:
                        outs[g - 2].wait()
                    destride_col(t % 2, c, obufs[g % 2], ln)
                    outs.append(pltpu.async_copy(
                        obufs[g % 2].at[pl.ds(0, ln)],
                        flat.at[pl.ds(c * stride + off, ln)], osem))
            for o in outs[-2:]:
                o.wait()

        def stage(tab, src_off, ln, flat, stride, dst_off):
            run_jobs([(tab, src_off, ln, flat, stride)])

        bbase = wid * BSTEPS * CW
        run_jobs([(btabt_hbm, bbase + s * CW, CW, bflat_out, NB)
                  for s in range(BSTEPS)])
        run_jobs([(ztabt_hbm, wid * ZW, ZW, zflat_out, NZ)])

        for i, (off, ln) in enumerate(B_EXTRA):
            @pl.when(wid == i)
            def _bextra(off=off, ln=ln):
                stage(btabt_hbm, off, ln, bflat_out, NB, off)
        for i, (off, ln) in enumerate(Z_EXTRA):
            @pl.when(wid == len(B_EXTRA) + i)
            def _zextra(off=off, ln=ln):
                stage(ztabt_hbm, off, ln, zflat_out, NZ, off)

        @pl.when(wid == len(B_EXTRA) + len(Z_EXTRA))
        def _tails():
            for c in range(ED):
                pltpu.async_copy(
                    btail_hbm.at[pl.ds(c * BT_RAG, BT_RAG)],
                    obuf0.at[pl.ds(0, BT_RAG)], isem).wait()
                pltpu.async_copy(
                    obuf0.at[pl.ds(0, BT_RAG)],
                    bflat_out.at[pl.ds(c * NB + NB - BT_RAG, BT_RAG)],
                    osem).wait()
                pltpu.async_copy(
                    ztail_hbm.at[pl.ds(c * ZT_RAG, ZT_RAG)],
                    obuf0.at[pl.ds(0, ZT_RAG)], isem).wait()
                pltpu.async_copy(
                    obuf0.at[pl.ds(0, ZT_RAG)],
                    zflat_out.at[pl.ds(c * NZ + NZ - ZT_RAG, ZT_RAG)],
                    osem).wait()

    return sc_flatten


def _make_sc_gather():
    mesh = plsc.VectorSubcoreMesh(
        core_axis_name="c", subcore_axis_name="s", num_cores=NC,
        num_subcores=NS)

    @functools.partial(
        pl.kernel,
        mesh=mesh,
        compiler_params=pltpu.CompilerParams(use_tc_tiling_on_sc=False),
        out_type=[
            jax.ShapeDtypeStruct((WPT // CHUNK, CHUNK), jnp.float32),
            jax.ShapeDtypeStruct((WPT // CHUNK, CHUNK), jnp.float32),
        ],
        scratch_types=[
            pltpu.VMEM((NCH, CHUNK), jnp.int32),
            pltpu.VMEM((NCH, CHUNK), jnp.int32),
            pltpu.VMEM((NCH, CHUNK), jnp.float32),
            pltpu.VMEM((NCH, CHUNK), jnp.float32),
            pltpu.SemaphoreType.DMA,
        ],
    )
    def sc_gather(bidx_hbm, zidx_hbm, btab_hbm, ztab_hbm, be_out, ze_out,
                  bidx_v, zidx_v, bw_v, zw_v, sem):
        wid = lax.axis_index("s") * NC + lax.axis_index("c")
        base = wid * NCH
        pltpu.sync_copy(bidx_hbm.at[pl.ds(base, NCH)], bidx_v)
        pltpu.sync_copy(zidx_hbm.at[pl.ds(base, NCH)], zidx_v)

        def step(g, carry):
            copies = []
            for j in range(GRP):
                c = g * GRP + j
                copies.append(
                    pltpu.async_copy(btab_hbm.at[bidx_v.at[c]], bw_v.at[c], sem))
                copies.append(
                    pltpu.async_copy(ztab_hbm.at[zidx_v.at[c]], zw_v.at[c], sem))
            for cp in copies:
                cp.wait()
            return carry

        lax.fori_loop(0, NSTEP, step, 0)
        pltpu.sync_copy(bw_v, be_out.at[pl.ds(base, NCH)])
        pltpu.sync_copy(zw_v, ze_out.at[pl.ds(base, NCH)])

    return sc_gather


def _mlp_body(beT_ref, zeT_ref, xT_ref, w1a_ref, w1b_ref, w1c_ref, b1_ref,
              w2_ref, b2_ref, w3_ref, b3_ref, o_ref):
    dn = (((0,), (0,)), ((), ()))
    h = (
        lax.dot_general(beT_ref[...], w1a_ref[...], dn,
                        preferred_element_type=jnp.float32)
        + lax.dot_general(zeT_ref[...], w1b_ref[...], dn,
                          preferred_element_type=jnp.float32)
        + lax.dot_general(xT_ref[...], w1c_ref[...], dn,
                          preferred_element_type=jnp.float32)
        + b1_ref[...]
    )
    h = jnp.maximum(h, 0.0)
    h = jnp.dot(h, w2_ref[...], preferred_element_type=jnp.float32) + b2_ref[...]
    h = jnp.maximum(h, 0.0)
    o_ref[...] = (
        jnp.dot(h, w3_ref[...], preferred_element_type=jnp.float32) + b3_ref[...]
    )


def kernel(brand_tensor, zip_tensor, input_tensor, brand_table, zip_table,
           W1, b1, W2, b2, W3, b3):
    btail = brand_table.T[:, NB - BT_RAG:].reshape(-1)
    ztail = zip_table.T[:, NZ - ZT_RAG:].reshape(-1)
    bflat, zflat = _make_sc_flatten()(brand_table.T, zip_table.T, btail,
                                      ztail)
    col = jnp.arange(ED, dtype=jnp.int32)
    bwidx = (col[:, None] * NB + brand_tensor[None, :]).reshape(
        WPT // CHUNK, CHUNK)
    zwidx = (col[:, None] * NZ + zip_tensor[None, :]).reshape(
        WPT // CHUNK, CHUNK)
    bew, zew = _make_sc_gather()(bwidx, zwidx, bflat, zflat)
    beT = bew.reshape(ED, B)
    zeT = zew.reshape(ED, B)
    xT = input_tensor.T

    w1a = W1[:ED]
    w1b = W1[ED:2 * ED]
    w1c = W1[2 * ED:]
    b1_2d = b1.reshape(1, -1)
    b2_2d = b2.reshape(1, -1)
    b3_2d = b3.reshape(1, -1)

    blk = 2048
    h1 = HD * 2
    out = pl.pallas_call(
        _mlp_body,
        grid=(B // blk,),
        in_specs=[
            pl.BlockSpec((ED, blk), lambda i: (0, i)),
            pl.BlockSpec((ED, blk), lambda i: (0, i)),
            pl.BlockSpec((IN_FEATURES, blk), lambda i: (0, i)),
            pl.BlockSpec((ED, h1), lambda i: (0, 0)),
            pl.BlockSpec((ED, h1), lambda i: (0, 0)),
            pl.BlockSpec((IN_FEATURES, h1), lambda i: (0, 0)),
            pl.BlockSpec((1, h1), lambda i: (0, 0)),
            pl.BlockSpec((h1, HD), lambda i: (0, 0)),
            pl.BlockSpec((1, HD), lambda i: (0, 0)),
            pl.BlockSpec((HD, 1), lambda i: (0, 0)),
            pl.BlockSpec((1, 1), lambda i: (0, 0)),
        ],
        out_specs=pl.BlockSpec((blk, 1), lambda i: (i, 0)),
        out_shape=jax.ShapeDtypeStruct((B, 1), jnp.float32),
    )(beT, zeT, xT, w1a, w1b, w1c, b1_2d, W2, b2_2d, W3, b3_2d)
    return out


# speedup vs baseline: 7.2048x; 1.0260x over previous
import functools

import jax
import jax.numpy as jnp
from jax import lax
from jax.experimental import pallas as pl
from jax.experimental.pallas import tpu as pltpu
from jax.experimental.pallas import tpu_sc as plsc

B = 16384
IN_FEATURES = 64
ED = 10
HD = 32
CHUNK = 128
NC = 2
NS = 16
NW = NC * NS


WPT = B * ED
NCH = WPT // (CHUNK * NW)
GRP = 4
NSTEP = NCH // GRP


NB = 1000000
NZ = 100000
WPT = B * ED
NCH = WPT // (CHUNK * NW)
GRP = 8
NSTEP = NCH // GRP
CW = 3840
BSTEPS = 8
B_EXTRA = ((983040, 3840), (986880, 3840), (990720, 3840), (994560, 3840),
           (998400, 1536))
ZW = 3072
Z_EXTRA = ((NW * ZW, 1664),)
BT_RAG = 64
ZT_RAG = 32


def _make_sc_flatten():
    mesh = plsc.VectorSubcoreMesh(
        core_axis_name="c", subcore_axis_name="s", num_cores=NC,
        num_subcores=NS)

    @functools.partial(
        pl.kernel,
        mesh=mesh,
        compiler_params=pltpu.CompilerParams(use_tc_tiling_on_sc=True),
        out_type=[
            jax.ShapeDtypeStruct((NB * ED,), jnp.float32),
            jax.ShapeDtypeStruct((NZ * ED,), jnp.float32),
        ],
        scratch_types=[
            pltpu.VMEM((2, ED, CW), jnp.float32),
            pltpu.VMEM((CW,), jnp.float32),
            pltpu.VMEM((CW,), jnp.float32),
            pltpu.SemaphoreType.DMA,
            pltpu.SemaphoreType.DMA,
        ],
    )
    def sc_flatten(btabt_hbm, ztabt_hbm, btail_hbm, ztail_hbm,
                   bflat_out, zflat_out, ibuf, obuf0, obuf1, isem, osem):
        wid = lax.axis_index("s") * NC + lax.axis_index("c")
        obufs = (obuf0, obuf1)

        def destride_col(slot, c, ob, ln):
            def vcopy(w, carry):
                ob[pl.ds(w * 128, 128)] = ibuf[slot, c, pl.ds(w * 128, 128)]
                return carry
            lax.fori_loop(0, ln // 128, vcopy, 0)

        def run_jobs(jobs):
            ins = [pltpu.async_copy(
                jobs[0][0].at[:, pl.ds(jobs[0][1], jobs[0][2])],
                ibuf.at[0].at[:, pl.ds(0, jobs[0][2])], isem)]
            outs = []
            for t, (tab, off, ln, flat, stride) in enumerate(jobs):
                if t + 1 < len(jobs):
                    nt, noff, nln = jobs[t + 1][0], jobs[t + 1][1], jobs[t + 1][2]
                    ins.append(pltpu.async_copy(
                        nt.at[:, pl.ds(noff, nln)],
                        ibuf.at[(t + 1) % 2].at[:, pl.ds(0, nln)], isem))
                ins[t].wait()
                for c in range(ED):
                    g = t * ED + c
                    if g >= 2:
                        outs[g - 2].wait()
                    destride_col(t % 2, c, obufs[g % 2], ln)
                    outs.append(pltpu.async_copy(
                        obufs[g % 2].at[pl.ds(0, ln)],
                        flat.at[pl.ds(c * stride + off, ln)], osem))
            for o in outs[-2:]:
                o.wait()

        def stage(tab, src_off, ln, flat, stride, dst_off):
            run_jobs([(tab, src_off, ln, flat, stride)])

        bbase = wid * BSTEPS * CW
        run_jobs([(btabt_hbm, bbase + s * CW, CW, bflat_out, NB)
                  for s in range(BSTEPS)])
        run_jobs([(ztabt_hbm, wid * ZW, ZW, zflat_out, NZ)])

        for i, (off, ln) in enumerate(B_EXTRA):
            @pl.when(wid == i)
            def _bextra(off=off, ln=ln):
                stage(btabt_hbm, off, ln, bflat_out, NB, off)
        for i, (off, ln) in enumerate(Z_EXTRA):
            @pl.when(wid == len(B_EXTRA) + i)
            def _zextra(off=off, ln=ln):
                stage(ztabt_hbm, off, ln, zflat_out, NZ, off)

        @pl.when(wid == len(B_EXTRA) + len(Z_EXTRA))
        def _tails():
            for c in range(ED):
                pltpu.async_copy(
                    btail_hbm.at[pl.ds(c * BT_RAG, BT_RAG)],
                    obuf0.at[pl.ds(0, BT_RAG)], isem).wait()
                pltpu.async_copy(
                    obuf0.at[pl.ds(0, BT_RAG)],
                    bflat_out.at[pl.ds(c * NB + NB - BT_RAG, BT_RAG)],
                    osem).wait()
                pltpu.async_copy(
                    ztail_hbm.at[pl.ds(c * ZT_RAG, ZT_RAG)],
                    obuf0.at[pl.ds(0, ZT_RAG)], isem).wait()
                pltpu.async_copy(
                    obuf0.at[pl.ds(0, ZT_RAG)],
                    zflat_out.at[pl.ds(c * NZ + NZ - ZT_RAG, ZT_RAG)],
                    osem).wait()

    return sc_flatten


def _make_sc_gather():
    mesh = plsc.VectorSubcoreMesh(
        core_axis_name="c", subcore_axis_name="s", num_cores=NC,
        num_subcores=NS)

    @functools.partial(
        pl.kernel,
        mesh=mesh,
        compiler_params=pltpu.CompilerParams(use_tc_tiling_on_sc=False),
        out_type=[
            jax.ShapeDtypeStruct((WPT // CHUNK, CHUNK), jnp.float32),
            jax.ShapeDtypeStruct((WPT // CHUNK, CHUNK), jnp.float32),
        ],
        scratch_types=[
            pltpu.VMEM((NCH, CHUNK), jnp.int32),
            pltpu.VMEM((NCH, CHUNK), jnp.int32),
            pltpu.VMEM((NCH, CHUNK), jnp.float32),
            pltpu.VMEM((NCH, CHUNK), jnp.float32),
            pltpu.SemaphoreType.DMA,
        ],
    )
    def sc_gather(bidx_hbm, zidx_hbm, btab_hbm, ztab_hbm, be_out, ze_out,
                  bidx_v, zidx_v, bw_v, zw_v, sem):
        wid = lax.axis_index("s") * NC + lax.axis_index("c")
        base = wid * NCH
        pltpu.sync_copy(bidx_hbm.at[pl.ds(base, NCH)], bidx_v)
        pltpu.sync_copy(zidx_hbm.at[pl.ds(base, NCH)], zidx_v)

        def step(g, carry):
            copies = []
            for j in range(GRP):
                c = g * GRP + j
                copies.append(
                    pltpu.async_copy(btab_hbm.at[bidx_v.at[c]], bw_v.at[c], sem))
                copies.append(
                    pltpu.async_copy(ztab_hbm.at[zidx_v.at[c]], zw_v.at[c], sem))
            for cp in copies:
                cp.wait()
            return carry

        lax.fori_loop(0, NSTEP, step, 0)
        pltpu.sync_copy(bw_v, be_out.at[pl.ds(base, NCH)])
        pltpu.sync_copy(zw_v, ze_out.at[pl.ds(base, NCH)])

    return sc_gather


def _mlp_body(beT_ref, zeT_ref, xT_ref, w1a_ref, w1b_ref, w1c_ref, b1_ref,
              w2_ref, b2_ref, w3_ref, b3_ref, o_ref):
    dn = (((0,), (0,)), ((), ()))
    h = (
        lax.dot_general(beT_ref[...], w1a_ref[...], dn,
                        preferred_element_type=jnp.float32)
        + lax.dot_general(zeT_ref[...], w1b_ref[...], dn,
                          preferred_element_type=jnp.float32)
        + lax.dot_general(xT_ref[...], w1c_ref[...], dn,
                          preferred_element_type=jnp.float32)
        + b1_ref[...]
    )
    h = jnp.maximum(h, 0.0)
    h = jnp.dot(h, w2_ref[...], preferred_element_type=jnp.float32) + b2_ref[...]
    h = jnp.maximum(h, 0.0)
    o_ref[...] = (
        jnp.dot(h, w3_ref[...], preferred_element_type=jnp.float32) + b3_ref[...]
    )


def kernel(brand_tensor, zip_tensor, input_tensor, brand_table, zip_table,
           W1, b1, W2, b2, W3, b3):
    btail = brand_table.T[:, NB - BT_RAG:].reshape(-1)
    ztail = zip_table.T[:, NZ - ZT_RAG:].reshape(-1)
    bflat, zflat = _make_sc_flatten()(brand_table.T, zip_table.T, btail,
                                      ztail)
    col = jnp.arange(ED, dtype=jnp.int32)
    bwidx = (col[:, None] * NB + brand_tensor[None, :]).reshape(
        WPT // CHUNK, CHUNK)
    zwidx = (col[:, None] * NZ + zip_tensor[None, :]).reshape(
        WPT // CHUNK, CHUNK)
    bew, zew = _make_sc_gather()(bwidx, zwidx, bflat, zflat)
    beT = bew.reshape(ED, B)
    zeT = zew.reshape(ED, B)
    xT = input_tensor.T

    w1a = W1[:ED]
    w1b = W1[ED:2 * ED]
    w1c = W1[2 * ED:]
    b1_2d = b1.reshape(1, -1)
    b2_2d = b2.reshape(1, -1)
    b3_2d = b3.reshape(1, -1)

    blk = 2048
    h1 = HD * 2
    out = pl.pallas_call(
        _mlp_body,
        grid=(B // blk,),
        in_specs=[
            pl.BlockSpec((ED, blk), lambda i: (0, i)),
            pl.BlockSpec((ED, blk), lambda i: (0, i)),
            pl.BlockSpec((IN_FEATURES, blk), lambda i: (0, i)),
            pl.BlockSpec((ED, h1), lambda i: (0, 0)),
            pl.BlockSpec((ED, h1), lambda i: (0, 0)),
            pl.BlockSpec((IN_FEATURES, h1), lambda i: (0, 0)),
            pl.BlockSpec((1, h1), lambda i: (0, 0)),
            pl.BlockSpec((h1, HD), lambda i: (0, 0)),
            pl.BlockSpec((1, HD), lambda i: (0, 0)),
            pl.BlockSpec((HD, 1), lambda i: (0, 0)),
            pl.BlockSpec((1, 1), lambda i: (0, 0)),
        ],
        out_specs=pl.BlockSpec((blk, 1), lambda i: (i, 0)),
        out_shape=jax.ShapeDtypeStruct((B, 1), jnp.float32),
    )(beT, zeT, xT, w1a, w1b, w1c, b1_2d, W2, b2_2d, W3, b3_2d)
    return out


# speedup vs baseline: 7.2174x; 1.0017x over previous
import functools

import jax
import jax.numpy as jnp
from jax import lax
from jax.experimental import pallas as pl
from jax.experimental.pallas import tpu as pltpu
from jax.experimental.pallas import tpu_sc as plsc

B = 16384
IN_FEATURES = 64
ED = 10
HD = 32
CHUNK = 128
NC = 2
NS = 16
NW = NC * NS


NB = 1000000
NZ = 100000
WPT = B * ED
NCH = WPT // (CHUNK * NW)
GRP = 8
NSTEP = NCH // GRP
CW = 3840
BSTEPS = 8
B_EXTRA = ((983040, 3840), (986880, 3840), (990720, 3840), (994560, 3840),
           (998400, 1536))
ZW = 3072
Z_EXTRA = ((NW * ZW, 1664),)
BT_RAG = 64
ZT_RAG = 32


def _make_sc_flatten():
    mesh = plsc.VectorSubcoreMesh(
        core_axis_name="c", subcore_axis_name="s", num_cores=NC,
        num_subcores=NS)

    @functools.partial(
        pl.kernel,
        mesh=mesh,
        compiler_params=pltpu.CompilerParams(use_tc_tiling_on_sc=True),
        out_type=[
            jax.ShapeDtypeStruct((NB * ED,), jnp.float32),
            jax.ShapeDtypeStruct((NZ * ED,), jnp.float32),
        ],
        scratch_types=[
            pltpu.VMEM((2, ED, CW), jnp.float32),
            pltpu.VMEM((CW,), jnp.float32),
            pltpu.VMEM((CW,), jnp.float32),
            pltpu.SemaphoreType.DMA,
            pltpu.SemaphoreType.DMA,
        ],
    )
    def sc_flatten(btabt_hbm, ztabt_hbm, btail_hbm, ztail_hbm,
                   bflat_out, zflat_out, ibuf, obuf0, obuf1, isem, osem):
        wid = lax.axis_index("s") * NC + lax.axis_index("c")
        obufs = (obuf0, obuf1)

        def destride_col(slot, c, ob, ln):
            def vcopy(w, carry):
                ob[pl.ds(w * 128, 128)] = ibuf[slot, c, pl.ds(w * 128, 128)]
                return carry
            lax.fori_loop(0, ln // 128, vcopy, 0)

        def run_jobs(jobs):
            ins = [pltpu.async_copy(
                jobs[0][0].at[:, pl.ds(jobs[0][1], jobs[0][2])],
                ibuf.at[0].at[:, pl.ds(0, jobs[0][2])], isem)]
            outs = []
            for t, (tab, off, ln, flat, stride) in enumerate(jobs):
                if t + 1 < len(jobs):
                    nt, noff, nln = jobs[t + 1][0], jobs[t + 1][1], jobs[t + 1][2]
                    ins.append(pltpu.async_copy(
                        nt.at[:, pl.ds(noff, nln)],
                        ibuf.at[(t + 1) % 2].at[:, pl.ds(0, nln)], isem))
                ins[t].wait()
                for c in range(ED):
                    g = t * ED + c
                    if g >= 2:
                        outs[g - 2].wait()
                    destride_col(t % 2, c, obufs[g % 2], ln)
                    outs.append(pltpu.async_copy(
                        obufs[g % 2].at[pl.ds(0, ln)],
                        flat.at[pl.ds(c * stride + off, ln)], osem))
            for o in outs[-2:]:
                o.wait()

        def stage(tab, src_off, ln, flat, stride, dst_off):
            run_jobs([(tab, src_off, ln, flat, stride)])

        bbase = wid * BSTEPS * CW
        run_jobs([(btabt_hbm, bbase + s * CW, CW, bflat_out, NB)
                  for s in range(BSTEPS)])
        run_jobs([(ztabt_hbm, wid * ZW, ZW, zflat_out, NZ)])

        for i, (off, ln) in enumerate(B_EXTRA):
            @pl.when(wid == i)
            def _bextra(off=off, ln=ln):
                stage(btabt_hbm, off, ln, bflat_out, NB, off)
        for i, (off, ln) in enumerate(Z_EXTRA):
            @pl.when(wid == len(B_EXTRA) + i)
            def _zextra(off=off, ln=ln):
                stage(ztabt_hbm, off, ln, zflat_out, NZ, off)

        @pl.when(wid == len(B_EXTRA) + len(Z_EXTRA))
        def _tails():
            for c in range(ED):
                pltpu.async_copy(
                    btail_hbm.at[pl.ds(c * BT_RAG, BT_RAG)],
                    obuf0.at[pl.ds(0, BT_RAG)], isem).wait()
                pltpu.async_copy(
                    obuf0.at[pl.ds(0, BT_RAG)],
                    bflat_out.at[pl.ds(c * NB + NB - BT_RAG, BT_RAG)],
                    osem).wait()
                pltpu.async_copy(
                    ztail_hbm.at[pl.ds(c * ZT_RAG, ZT_RAG)],
                    obuf0.at[pl.ds(0, ZT_RAG)], isem).wait()
                pltpu.async_copy(
                    obuf0.at[pl.ds(0, ZT_RAG)],
                    zflat_out.at[pl.ds(c * NZ + NZ - ZT_RAG, ZT_RAG)],
                    osem).wait()

    return sc_flatten


def _make_sc_gather():
    mesh = plsc.VectorSubcoreMesh(
        core_axis_name="c", subcore_axis_name="s", num_cores=NC,
        num_subcores=NS)

    @functools.partial(
        pl.kernel,
        mesh=mesh,
        compiler_params=pltpu.CompilerParams(use_tc_tiling_on_sc=False),
        out_type=[
            jax.ShapeDtypeStruct((WPT // CHUNK, CHUNK), jnp.float32),
            jax.ShapeDtypeStruct((WPT // CHUNK, CHUNK), jnp.float32),
        ],
        scratch_types=[
            pltpu.VMEM((NCH, CHUNK), jnp.int32),
            pltpu.VMEM((NCH, CHUNK), jnp.int32),
            pltpu.VMEM((NCH, CHUNK), jnp.float32),
            pltpu.VMEM((NCH, CHUNK), jnp.float32),
            pltpu.SemaphoreType.DMA,
        ],
    )
    def sc_gather(bidx_hbm, zidx_hbm, btab_hbm, ztab_hbm, be_out, ze_out,
                  bidx_v, zidx_v, bw_v, zw_v, sem):
        wid = lax.axis_index("s") * NC + lax.axis_index("c")
        base = wid * NCH
        pltpu.sync_copy(bidx_hbm.at[pl.ds(base, NCH)], bidx_v)
        pltpu.sync_copy(zidx_hbm.at[pl.ds(base, NCH)], zidx_v)

        def step(g, carry):
            copies = []
            for j in range(GRP):
                c = g * GRP + j
                copies.append(
                    pltpu.async_copy(btab_hbm.at[bidx_v.at[c]], bw_v.at[c], sem))
                copies.append(
                    pltpu.async_copy(ztab_hbm.at[zidx_v.at[c]], zw_v.at[c], sem))
            for cp in copies:
                cp.wait()
            return carry

        lax.fori_loop(0, NSTEP, step, 0)
        pltpu.sync_copy(bw_v, be_out.at[pl.ds(base, NCH)])
        pltpu.sync_copy(zw_v, ze_out.at[pl.ds(base, NCH)])

    return sc_gather


def _mlp_body(beT_ref, zeT_ref, xT_ref, w1a_ref, w1b_ref, w1c_ref, b1_ref,
              w2_ref, b2_ref, w3_ref, b3_ref, o_ref):
    dn = (((0,), (0,)), ((), ()))
    h = (
        lax.dot_general(beT_ref[...], w1a_ref[...], dn,
                        preferred_element_type=jnp.float32)
        + lax.dot_general(zeT_ref[...], w1b_ref[...], dn,
                          preferred_element_type=jnp.float32)
        + lax.dot_general(xT_ref[...], w1c_ref[...], dn,
                          preferred_element_type=jnp.float32)
        + b1_ref[...]
    )
    h = jnp.maximum(h, 0.0)
    h = jnp.dot(h, w2_ref[...], preferred_element_type=jnp.float32) + b2_ref[...]
    h = jnp.maximum(h, 0.0)
    o_ref[...] = (
        jnp.dot(h, w3_ref[...], preferred_element_type=jnp.float32) + b3_ref[...]
    )


def kernel(brand_tensor, zip_tensor, input_tensor, brand_table, zip_table,
           W1, b1, W2, b2, W3, b3):
    btail = brand_table.T[:, NB - BT_RAG:].reshape(-1)
    ztail = zip_table.T[:, NZ - ZT_RAG:].reshape(-1)
    bflat, zflat = _make_sc_flatten()(brand_table.T, zip_table.T, btail,
                                      ztail)
    col = jnp.arange(ED, dtype=jnp.int32)
    bwidx = (col[:, None] * NB + brand_tensor[None, :]).reshape(
        WPT // CHUNK, CHUNK)
    zwidx = (col[:, None] * NZ + zip_tensor[None, :]).reshape(
        WPT // CHUNK, CHUNK)
    bew, zew = _make_sc_gather()(bwidx, zwidx, bflat, zflat)
    beT = bew.reshape(ED, B)
    zeT = zew.reshape(ED, B)
    xT = input_tensor.T

    w1a = W1[:ED]
    w1b = W1[ED:2 * ED]
    w1c = W1[2 * ED:]
    b1_2d = b1.reshape(1, -1)
    b2_2d = b2.reshape(1, -1)
    b3_2d = b3.reshape(1, -1)

    blk = 2048
    h1 = HD * 2
    out = pl.pallas_call(
        _mlp_body,
        grid=(B // blk,),
        in_specs=[
            pl.BlockSpec((ED, blk), lambda i: (0, i)),
            pl.BlockSpec((ED, blk), lambda i: (0, i)),
            pl.BlockSpec((IN_FEATURES, blk), lambda i: (0, i)),
            pl.BlockSpec((ED, h1), lambda i: (0, 0)),
            pl.BlockSpec((ED, h1), lambda i: (0, 0)),
            pl.BlockSpec((IN_FEATURES, h1), lambda i: (0, 0)),
            pl.BlockSpec((1, h1), lambda i: (0, 0)),
            pl.BlockSpec((h1, HD), lambda i: (0, 0)),
            pl.BlockSpec((1, HD), lambda i: (0, 0)),
            pl.BlockSpec((HD, 1), lambda i: (0, 0)),
            pl.BlockSpec((1, 1), lambda i: (0, 0)),
        ],
        out_specs=pl.BlockSpec((blk, 1), lambda i: (i, 0)),
        out_shape=jax.ShapeDtypeStruct((B, 1), jnp.float32),
    )(beT, zeT, xT, w1a, w1b, w1c, b1_2d, W2, b2_2d, W3, b3_2d)
    return out
